# Initial kernel scaffold; baseline (speedup 1.0000x reference)
import jax
import jax.numpy as jnp
from jax.experimental import pallas as pl


def kernel(feature_map, agent_positions, mask):
    raise NotImplementedError("write your pallas kernel here")


# speedup vs baseline: 1.0600x; 1.0600x over previous
import functools

import jax
import jax.numpy as jnp
from jax import lax
from jax.experimental import pallas as pl
from jax.experimental.pallas import tpu as pltpu
from jax.experimental.pallas import tpu_sc as plsc

T, H, W, C, A = 128, 64, 64, 128, 64
B = T * A
NC, NS, L = 2, 16, 16
NW = NC * NS
BPW = B // NW
NIDX = BPW // 128


def _make_sc_gather():
    mesh = plsc.VectorSubcoreMesh(core_axis_name="c", subcore_axis_name="s")

    @functools.partial(
        pl.kernel,
        out_type=jax.ShapeDtypeStruct((B, C), jnp.float32),
        mesh=mesh,
        scratch_types=[
            pltpu.VMEM((BPW,), jnp.int32),
            pltpu.VMEM((BPW,), jnp.int32),
            pltpu.VMEM((BPW,), jnp.int32),
            pltpu.VMEM((NIDX, 128), jnp.int32),
            pltpu.VMEM((BPW, C), jnp.float32),
            pltpu.SemaphoreType.DMA,
        ],
    )
    def gather_kernel(table, rows, cols, mask, out, r_v, c_v, m_v, idx_v,
                      feat_v, sem):
        wid = lax.axis_index("s") * NC + lax.axis_index("c")
        base = wid * BPW

        pltpu.sync_copy(rows.at[pl.ds(base, BPW)], r_v)
        pltpu.sync_copy(cols.at[pl.ds(base, BPW)], c_v)
        pltpu.sync_copy(mask.at[pl.ds(base, BPW)], m_v)

        lane = lax.iota(jnp.int32, L)
        for k in range(BPW // L):
            g = base + (k * L + lane)
            t = lax.shift_right_logical(g, 6)
            r = r_v[pl.ds(k * L, L)]
            c = c_v[pl.ds(k * L, L)]
            flat = t * (H * W) + r * W + c
            idx_v[k * L // 128, pl.ds((k * L) % 128, L)] = flat

        copies = [
            pltpu.async_copy(
                table.at[idx_v.at[j]], feat_v.at[pl.ds(j * 128, 128)], sem)
            for j in range(NIDX)
        ]
        for cp in copies:
            cp.wait()

        def zero_group(g16, _):
            mv = m_v[pl.ds(g16 * L, L)].astype(jnp.float32)
            for j in range(L):
                mrow = jnp.broadcast_to(lax.slice(mv, (j,), (j + 1,)), (L,))
                row = g16 * L + j
                for cc in range(C // L):
                    feat_v[row, pl.ds(cc * L, L)] = (
                        feat_v[row, pl.ds(cc * L, L)] * mrow)
            return 0

        lax.fori_loop(0, BPW // L, zero_group, 0)

        pltpu.sync_copy(feat_v, out.at[pl.ds(base, BPW)])

    return gather_kernel


_sc_gather = _make_sc_gather()


def kernel(feature_map, agent_positions, mask):
    table = feature_map.reshape(T * H * W, C)
    rows = agent_positions[..., 0].reshape(B)
    cols = agent_positions[..., 1].reshape(B)
    mask_i = mask.reshape(B).astype(jnp.int32)
    out = _sc_gather(table, rows, cols, mask_i)
    return out.reshape(T, A, C)


# speedup vs baseline: 1.1530x; 1.0878x over previous
import functools

import jax
import jax.numpy as jnp
from jax import lax
from jax.experimental import pallas as pl
from jax.experimental.pallas import tpu as pltpu
from jax.experimental.pallas import tpu_sc as plsc

T, H, W, C, A = 128, 64, 64, 128, 64
B = T * A
NC, NS, L = 2, 16, 16
NW = NC * NS
BPW = B // NW
NQ = 4
QR = BPW // NQ


def _make_sc_gather():
    mesh = plsc.VectorSubcoreMesh(core_axis_name="c", subcore_axis_name="s")

    @functools.partial(
        pl.kernel,
        out_type=jax.ShapeDtypeStruct((B, C), jnp.float32),
        mesh=mesh,
        scratch_types=[
            pltpu.VMEM((3 * BPW,), jnp.int32),
            pltpu.VMEM((NQ, QR), jnp.int32),
            pltpu.VMEM((BPW, C), jnp.float32),
            pltpu.SemaphoreType.DMA,
            pltpu.SemaphoreType.DMA,
            pltpu.SemaphoreType.DMA,
            pltpu.SemaphoreType.DMA,
            pltpu.SemaphoreType.DMA,
        ],
    )
    def gather_kernel(table, packed, out, p_v, idx_v, feat_v,
                      g0, g1, g2, g3, wsem):
        gsems = (g0, g1, g2, g3)
        wid = lax.axis_index("s") * NC + lax.axis_index("c")
        base = wid * BPW

        pltpu.sync_copy(packed.at[wid], p_v)

        lane = lax.iota(jnp.int32, L)
        for k in range(BPW // L):
            g = base + (k * L + lane)
            t = lax.shift_right_logical(g, 6)
            r = p_v[pl.ds(k * L, L)]
            c = p_v[pl.ds(BPW + k * L, L)]
            flat = t * (H * W) + r * W + c
            idx_v[k * L // QR, pl.ds((k * L) % QR, L)] = flat

        copies = [
            pltpu.async_copy(
                table.at[idx_v.at[q]], feat_v.at[pl.ds(q * QR, QR)], gsems[q])
            for q in range(NQ)
        ]

        wcopies = []
        for q in range(NQ):
            copies[q].wait()

            def mul_group(g16, _, q=q):
                mv = p_v[pl.ds(2 * BPW + q * QR + g16 * L, L)]
                mvf = mv.astype(jnp.float32)
                for j in range(L):
                    mrow = jnp.broadcast_to(
                        lax.slice(mvf, (j,), (j + 1,)), (L,))
                    row = q * QR + g16 * L + j
                    for cc in range(C // L):
                        feat_v[row, pl.ds(cc * L, L)] = (
                            feat_v[row, pl.ds(cc * L, L)] * mrow)
                return 0

            lax.fori_loop(0, QR // L, mul_group, 0)
            wcopies.append(pltpu.async_copy(
                feat_v.at[pl.ds(q * QR, QR)],
                out.at[pl.ds(base + q * QR, QR)], wsem))

        for wc in wcopies:
            wc.wait()

    return gather_kernel


_sc_gather = _make_sc_gather()


def kernel(feature_map, agent_positions, mask):
    table = feature_map.reshape(T * H * W, C)
    rows = agent_positions[..., 0].reshape(NW, BPW)
    cols = agent_positions[..., 1].reshape(NW, BPW)
    mask_i = mask.reshape(NW, BPW).astype(jnp.int32)
    packed = jnp.concatenate([rows, cols, mask_i], axis=1)
    out = _sc_gather(table, packed)
    return out.reshape(T, A, C)
